# Initial kernel scaffold; baseline (speedup 1.0000x reference)
import jax
import jax.numpy as jnp
from jax.experimental import pallas as pl


def kernel(x, zones_ids, lf_W1, lf_b1, lf_W2, lf_b2, W1, b1, W2, b2, W3, b3):
    raise NotImplementedError("write your pallas kernel here")


# speedup vs baseline: 40.2603x; 40.2603x over previous
import functools

import jax
import jax.numpy as jnp
from jax import lax
from jax.experimental import pallas as pl
from jax.experimental.pallas import tpu as pltpu
from jax.experimental.pallas import tpu_sc as plsc

_F32 = jnp.float32
_I32 = jnp.int32


def _prep_body(x_ref, z_ref, wa_ref, ba_ref, wb_ref, bb_ref, w1a_ref,
               w1z_ref, w1px_ref, w1py_ref, local_ref, a1_ref):
    xr = x_ref[0]
    px = xr[:, 0:1]
    py = xr[:, 1:2]
    zc = z_ref[0]
    h = jnp.tanh(px * wa_ref[0:1] + py * wa_ref[1:2] + ba_ref[...])
    local = jnp.tanh(
        jnp.dot(h, wb_ref[...], preferred_element_type=_F32) + bb_ref[...])
    a1 = (jnp.dot(local, w1a_ref[...], preferred_element_type=_F32)
          + zc * w1z_ref[...] + px * w1px_ref[...] + py * w1py_ref[...])
    local_ref[0] = local
    a1_ref[0] = jnp.concatenate([a1, jnp.zeros_like(a1)], axis=1)


def _prep(x, z, lfW1, lfb1, lfW2, lfb2, w1a, w1z, w1px, w1py):
    B, N, _ = x.shape
    bs3 = lambda d: pl.BlockSpec((1, N, d), lambda i: (i, 0, 0))
    ws = lambda s: pl.BlockSpec(s, lambda i: (0,) * len(s))
    return pl.pallas_call(
        _prep_body,
        grid=(B,),
        in_specs=[bs3(2), bs3(1), ws((2, 64)), ws((1, 64)), ws((64, 64)),
                  ws((1, 64)), ws((64, 64)), ws((1, 64)), ws((1, 64)),
                  ws((1, 64))],
        out_specs=[bs3(64), bs3(128)],
        out_shape=[jax.ShapeDtypeStruct((B, N, 64), _F32),
                   jax.ShapeDtypeStruct((B, N, 128), _F32)],
    )(x, z, lfW1, lfb1, lfW2, lfb2, w1a, w1z, w1px, w1py)


def _fps_body(n_s, px_ref, py_ref, qx_ref, qy_ref):
    B, N = px_ref.shape
    px = px_ref[...]
    py = py_ref[...]
    iota = lax.broadcasted_iota(_I32, (B, N), 1)
    iota_s = lax.broadcasted_iota(_I32, (B, n_s), 1)

    def step(i, carry):
        dists, fidx, qxa, qya = carry
        onehot = iota == fidx
        fx = jnp.sum(jnp.where(onehot, px, 0.0), axis=1, keepdims=True)
        fy = jnp.sum(jnp.where(onehot, py, 0.0), axis=1, keepdims=True)
        slot = iota_s == i
        qxa = jnp.where(slot, fx, qxa)
        qya = jnp.where(slot, fy, qya)
        dx = px - fx
        dy = py - fy
        dists = jnp.minimum(dists, dx * dx + dy * dy)
        m = jnp.max(dists, axis=1, keepdims=True)
        cand = jnp.where(dists == m, iota, N)
        fidx = jnp.min(cand, axis=1, keepdims=True)
        return dists, fidx, qxa, qya

    _, _, qxa, qya = lax.fori_loop(
        0, n_s, step,
        (jnp.full((B, N), 1e10, _F32), jnp.zeros((B, 1), _I32),
         jnp.zeros((B, n_s), _F32), jnp.zeros((B, n_s), _F32)))
    qx_ref[...] = qxa
    qy_ref[...] = qya


def _fps(px, py, n_s):
    B, N = px.shape
    return pl.pallas_call(
        functools.partial(_fps_body, n_s),
        in_specs=[pl.BlockSpec((B, N), lambda: (0, 0))] * 2,
        out_specs=[pl.BlockSpec((B, n_s), lambda: (0, 0))] * 2,
        out_shape=[jax.ShapeDtypeStruct((B, n_s), _F32),
                   jax.ShapeDtypeStruct((B, n_s), _F32)],
    )(px, py)


def _make_ballmax(B, Nsrc, Nq, F, r2, Ftab):
    NC, NS = 2, 16
    NW = NC * NS
    WPB = NW // B
    QW = Nq // WPB
    CH = Nsrc // 16
    K = 128
    NFV = F // 16
    mesh = plsc.VectorSubcoreMesh(core_axis_name="c", subcore_axis_name="s",
                                  num_cores=NC, num_subcores=NS)

    @functools.partial(
        pl.kernel,
        out_type=jax.ShapeDtypeStruct((B * Nq, F), _F32),
        mesh=mesh,
        compiler_params=pltpu.CompilerParams(needs_layout_passes=False),
        scratch_types=[
            pltpu.VMEM((Nsrc,), _F32), pltpu.VMEM((Nsrc,), _F32),
            pltpu.VMEM((QW,), _F32), pltpu.VMEM((QW,), _F32),
            pltpu.VMEM((K,), _I32), pltpu.VMEM((K, Ftab), _F32),
            pltpu.VMEM((QW, F), _F32),
            pltpu.SemaphoreType.DMA,
        ],
    )
    def ballmax(sx_hbm, sy_hbm, qx_hbm, qy_hbm, a_hbm, out_hbm,
                sxv, syv, qxv, qyv, idxv, rowsv, outv, sem):
        c = lax.axis_index("c")
        s = lax.axis_index("s")
        w = s * NC + c
        b = w // WPB
        qoff = (w % WPB) * QW
        pltpu.sync_copy(sx_hbm.at[b], sxv)
        pltpu.sync_copy(sy_hbm.at[b], syv)
        pltpu.sync_copy(qx_hbm.at[b, pl.ds(qoff, QW)], qxv)
        pltpu.sync_copy(qy_hbm.at[b, pl.ds(qoff, QW)], qyv)
        base = b * Nsrc
        lane = lax.broadcasted_iota(_I32, (16,), 0)

        def per_query(q, carry):
            qsplat = jnp.full((16,), q, _I32)
            qxs = plsc.load_gather(qxv, [qsplat])
            qys = plsc.load_gather(qyv, [qsplat])

            def cond(st):
                ch, cnt = st
                return jnp.logical_and(ch < CH, cnt < K)

            def body(st):
                ch, cnt = st
                off = ch * 16
                sxc = sxv[pl.ds(off, 16)]
                syc = syv[pl.ds(off, 16)]
                dx = sxc - qxs
                dy = syc - qys
                d2 = dx * dx + dy * dy
                m = d2 <= r2
                pc = plsc.cumsum(m.astype(_I32))
                keep = jnp.logical_and(m, pc <= (K - cnt))
                slots = pc + (cnt - 1)
                gidx = lane + (off + base)
                plsc.store_scatter(idxv, [slots], gidx, mask=keep)
                inc = jnp.max(jnp.where(keep, pc, 0))
                return ch + jnp.array(1, _I32), cnt + inc

            _, cnt = lax.while_loop(
                cond, body, (jnp.array(0, _I32), jnp.array(0, _I32)))

            v0 = plsc.load_gather(idxv, [jnp.zeros((16,), _I32)])
            for c8 in range(K // 16):
                posn = lane + (c8 * 16)
                plsc.store_scatter(idxv, [posn], v0, mask=posn >= cnt)

            pltpu.async_copy(a_hbm.at[idxv], rowsv, sem).wait()

            def acc_step(k, accs):
                return tuple(
                    jnp.maximum(accs[f], rowsv[k, pl.ds(f * 16, 16)])
                    for f in range(NFV))

            accs = lax.fori_loop(
                0, K, acc_step,
                tuple(jnp.full((16,), -3.0e38, _F32) for _ in range(NFV)))
            for f in range(NFV):
                outv[q, pl.ds(f * 16, 16)] = accs[f]
            return carry

        lax.fori_loop(0, QW, per_query, jnp.array(0, _I32))
        pltpu.sync_copy(outv, out_hbm.at[pl.ds(w * QW, QW)])

    return ballmax


def _mid_body(sm_ref, qx_ref, qy_ref, b1_ref, w1px_ref, w1py_ref,
              w2a_ref, w2px_ref, w2py_ref, a2_ref):
    sm = sm_ref[0]
    qx = qx_ref[0]
    qy = qy_ref[0]
    x1 = sm + b1_ref[...] - qx * w1px_ref[...] - qy * w1py_ref[...]
    a2 = (jnp.dot(x1, w2a_ref[...], preferred_element_type=_F32)
          + qx * w2px_ref[...] + qy * w2py_ref[...])
    a2_ref[0] = a2


def _mid(sm, qx, qy, b1, w1px, w1py, w2a, w2px, w2py):
    B, Nq, _ = sm.shape
    bs3 = lambda d: pl.BlockSpec((1, Nq, d), lambda i: (i, 0, 0))
    ws = lambda s: pl.BlockSpec(s, lambda i: (0,) * len(s))
    return pl.pallas_call(
        _mid_body,
        grid=(B,),
        in_specs=[bs3(64), bs3(1), bs3(1), ws((1, 64)), ws((1, 64)),
                  ws((1, 64)), ws((64, 128)), ws((1, 128)), ws((1, 128))],
        out_specs=bs3(128),
        out_shape=jax.ShapeDtypeStruct((B, Nq, 128), _F32),
    )(sm, qx, qy, b1, w1px, w1py, w2a, w2px, w2py)


def _fin_body(sm_ref, qx_ref, qy_ref, b2_ref, w2px_ref, w2py_ref,
              w3a_ref, w3px_ref, w3py_ref, b3_ref, g_ref):
    sm = sm_ref[0]
    qx = qx_ref[0]
    qy = qy_ref[0]
    x2 = sm + b2_ref[...] - qx * w2px_ref[...] - qy * w2py_ref[...]
    g = (jnp.dot(x2, w3a_ref[...], preferred_element_type=_F32)
         + qx * w3px_ref[...] + qy * w3py_ref[...] + b3_ref[...])
    g_ref[0] = jnp.max(g, axis=0, keepdims=True)


def _fin(sm, qx, qy, b2, w2px, w2py, w3a, w3px, w3py, b3):
    B, Nq, _ = sm.shape
    bs3 = lambda d: pl.BlockSpec((1, Nq, d), lambda i: (i, 0, 0))
    ws = lambda s: pl.BlockSpec(s, lambda i: (0,) * len(s))
    return pl.pallas_call(
        _fin_body,
        grid=(B,),
        in_specs=[bs3(128), bs3(1), bs3(1), ws((1, 128)), ws((1, 128)),
                  ws((1, 128)), ws((128, 1024)), ws((1, 1024)),
                  ws((1, 1024)), ws((1, 1024))],
        out_specs=pl.BlockSpec((1, 1, 1024), lambda i: (i, 0, 0)),
        out_shape=jax.ShapeDtypeStruct((B, 1, 1024), _F32),
    )(sm, qx, qy, b2, w2px, w2py, w3a, w3px, w3py, b3)


def kernel(x, zones_ids, lf_W1, lf_b1, lf_W2, lf_b2, W1, b1, W2, b2, W3, b3):
    B, N, _ = x.shape
    n1 = N // 2
    n2 = n1 // 4
    px = x[:, :, 0]
    py = x[:, :, 1]
    r1 = lambda v: v.reshape(1, -1)

    local, a1 = _prep(x, zones_ids, lf_W1, r1(lf_b1), lf_W2, r1(lf_b2),
                      W1[:64], r1(W1[64]), r1(W1[65]), r1(W1[66]))
    qx1, qy1 = _fps(px, py, n1)
    sm1 = _make_ballmax(B, N, n1, 64, 0.25, 128)(
        px, py, qx1, qy1, a1.reshape(B * N, 128))
    a2 = _mid(sm1.reshape(B, n1, 64), qx1[..., None], qy1[..., None],
              r1(b1), r1(W1[65]), r1(W1[66]), W2[:64], r1(W2[64]), r1(W2[65]))
    qx2, qy2 = _fps(qx1, qy1, n2)
    sm2 = _make_ballmax(B, n1, n2, 128, 1.0, 128)(
        qx1, qy1, qx2, qy2, a2.reshape(B * n1, 128))
    gfeat = _fin(sm2.reshape(B, n2, 128), qx2[..., None], qy2[..., None],
                 r1(b2), r1(W2[64]), r1(W2[65]), W3[:128], r1(W3[128]),
                 r1(W3[129]), r1(b3))
    return local, gfeat.reshape(B, 1024)


# speedup vs baseline: 41.5688x; 1.0325x over previous
import functools

import jax
import jax.numpy as jnp
from jax import lax
from jax.experimental import pallas as pl
from jax.experimental.pallas import tpu as pltpu
from jax.experimental.pallas import tpu_sc as plsc

_F32 = jnp.float32
_I32 = jnp.int32


def _prep_body(x_ref, z_ref, wa_ref, ba_ref, wb_ref, bb_ref, w1a_ref,
               w1z_ref, w1px_ref, w1py_ref, local_ref, a1_ref):
    xr = x_ref[0]
    px = xr[:, 0:1]
    py = xr[:, 1:2]
    zc = z_ref[0]
    h = jnp.tanh(px * wa_ref[0:1] + py * wa_ref[1:2] + ba_ref[...])
    local = jnp.tanh(
        jnp.dot(h, wb_ref[...], preferred_element_type=_F32) + bb_ref[...])
    a1 = (jnp.dot(local, w1a_ref[...], preferred_element_type=_F32)
          + zc * w1z_ref[...] + px * w1px_ref[...] + py * w1py_ref[...])
    local_ref[0] = local
    a1_ref[0] = jnp.concatenate([a1, jnp.zeros_like(a1)], axis=1)


def _prep(x, z, lfW1, lfb1, lfW2, lfb2, w1a, w1z, w1px, w1py):
    B, N, _ = x.shape
    bs3 = lambda d: pl.BlockSpec((1, N, d), lambda i: (i, 0, 0))
    ws = lambda s: pl.BlockSpec(s, lambda i: (0,) * len(s))
    return pl.pallas_call(
        _prep_body,
        grid=(B,),
        in_specs=[bs3(2), bs3(1), ws((2, 64)), ws((1, 64)), ws((64, 64)),
                  ws((1, 64)), ws((64, 64)), ws((1, 64)), ws((1, 64)),
                  ws((1, 64))],
        out_specs=[bs3(64), bs3(128)],
        out_shape=[jax.ShapeDtypeStruct((B, N, 64), _F32),
                   jax.ShapeDtypeStruct((B, N, 128), _F32)],
    )(x, z, lfW1, lfb1, lfW2, lfb2, w1a, w1z, w1px, w1py)


def _fps_body(n_s, px_ref, py_ref, qx_ref, qy_ref):
    B, N = px_ref.shape
    px = px_ref[...]
    py = py_ref[...]
    iota = lax.broadcasted_iota(_I32, (B, N), 1)
    iota_b = lax.broadcasted_iota(_I32, (B, 128), 1)

    def step(t, carry):
        dists, fidx, qxb, qyb = carry
        onehot = iota == fidx
        fx = jnp.sum(jnp.where(onehot, px, 0.0), axis=1, keepdims=True)
        fy = jnp.sum(jnp.where(onehot, py, 0.0), axis=1, keepdims=True)
        slot = iota_b == t
        qxb = jnp.where(slot, fx, qxb)
        qyb = jnp.where(slot, fy, qyb)
        dx = px - fx
        dy = py - fy
        dists = jnp.minimum(dists, dx * dx + dy * dy)
        m = jnp.max(dists, axis=1, keepdims=True)
        cand = jnp.where(dists == m, iota, N)
        fidx = jnp.min(cand, axis=1, keepdims=True)
        return dists, fidx, qxb, qyb

    dists = jnp.full((B, N), 1e10, _F32)
    fidx = jnp.zeros((B, 1), _I32)
    zb = jnp.zeros((B, 128), _F32)
    for blk in range(n_s // 128):
        dists, fidx, qxb, qyb = lax.fori_loop(
            0, 128, step, (dists, fidx, zb, zb))
        qx_ref[:, blk * 128:(blk + 1) * 128] = qxb
        qy_ref[:, blk * 128:(blk + 1) * 128] = qyb


def _fps(px, py, n_s):
    B, N = px.shape
    return pl.pallas_call(
        functools.partial(_fps_body, n_s),
        in_specs=[pl.BlockSpec((B, N), lambda: (0, 0))] * 2,
        out_specs=[pl.BlockSpec((B, n_s), lambda: (0, 0))] * 2,
        out_shape=[jax.ShapeDtypeStruct((B, n_s), _F32),
                   jax.ShapeDtypeStruct((B, n_s), _F32)],
    )(px, py)


def _make_ballmax(B, Nsrc, Nq, F, r2, Ftab):
    NC, NS = 2, 16
    NW = NC * NS
    WPB = NW // B
    QW = Nq // WPB
    CH = Nsrc // 16
    K = 128
    NFV = F // 16
    mesh = plsc.VectorSubcoreMesh(core_axis_name="c", subcore_axis_name="s",
                                  num_cores=NC, num_subcores=NS)

    @functools.partial(
        pl.kernel,
        out_type=jax.ShapeDtypeStruct((B * Nq, F), _F32),
        mesh=mesh,
        compiler_params=pltpu.CompilerParams(needs_layout_passes=False),
        scratch_types=[
            pltpu.VMEM((Nsrc,), _F32), pltpu.VMEM((Nsrc,), _F32),
            pltpu.VMEM((QW,), _F32), pltpu.VMEM((QW,), _F32),
            pltpu.VMEM((2, K), _I32), pltpu.VMEM((2, K, Ftab), _F32),
            pltpu.VMEM((QW, F), _F32),
            pltpu.SemaphoreType.DMA, pltpu.SemaphoreType.DMA,
        ],
    )
    def ballmax(sx_hbm, sy_hbm, qx_hbm, qy_hbm, a_hbm, out_hbm,
                sxv, syv, qxv, qyv, idx2, rows2, outv, sema, semb):
        c = lax.axis_index("c")
        s = lax.axis_index("s")
        w = s * NC + c
        b = w // WPB
        qoff = (w % WPB) * QW
        pltpu.sync_copy(sx_hbm.at[b], sxv)
        pltpu.sync_copy(sy_hbm.at[b], syv)
        pltpu.sync_copy(qx_hbm.at[b, pl.ds(qoff, QW)], qxv)
        pltpu.sync_copy(qy_hbm.at[b, pl.ds(qoff, QW)], qyv)
        base = b * Nsrc
        lane = lax.broadcasted_iota(_I32, (16,), 0)
        sems = (sema, semb)

        def fire_dma(p):
            pltpu.async_copy(a_hbm.at[idx2.at[p]], rows2.at[p], sems[p])

        def scan_query(q, p):
            idxr = idx2.at[p]
            qsplat = jnp.full((16,), q, _I32)
            qxs = plsc.load_gather(qxv, [qsplat])
            qys = plsc.load_gather(qyv, [qsplat])

            def cond(st):
                ch, cnt = st
                return jnp.logical_and(ch < CH, cnt < K)

            def body(st):
                ch, cnt = st
                off = ch * 16
                sxc = sxv[pl.ds(off, 16)]
                syc = syv[pl.ds(off, 16)]
                dx = sxc - qxs
                dy = syc - qys
                d2 = dx * dx + dy * dy
                m = d2 <= r2
                pc = plsc.cumsum(m.astype(_I32))
                keep = jnp.logical_and(m, pc <= (K - cnt))
                slots = pc + (cnt - 1)
                gidx = lane + (off + base)
                plsc.store_scatter(idxr, [slots], gidx, mask=keep)
                inc = jnp.max(jnp.where(keep, pc, 0))
                return ch + jnp.array(1, _I32), cnt + inc

            _, cnt = lax.while_loop(
                cond, body, (jnp.array(0, _I32), jnp.array(0, _I32)))

            @pl.when(cnt < K)
            def _pad():
                v0 = plsc.load_gather(idxr, [jnp.zeros((16,), _I32)])
                for c8 in range(K // 16):
                    posn = lane + (c8 * 16)
                    plsc.store_scatter(idxr, [posn], v0, mask=posn >= cnt)

        def wait_dma(p):
            pltpu.make_async_copy(a_hbm.at[idx2.at[p]], rows2.at[p],
                                  sems[p]).wait()

        def acc_out(q, p):
            rr = rows2.at[p]

            def acc_step(k8, accs):
                base8 = k8 * 8
                for u in range(8):
                    accs = tuple(
                        jnp.maximum(accs[f], rr[base8 + u, pl.ds(f * 16, 16)])
                        for f in range(NFV))
                return accs

            accs = lax.fori_loop(
                0, K // 8, acc_step,
                tuple(jnp.full((16,), -3.0e38, _F32) for _ in range(NFV)))
            for f in range(NFV):
                outv[q, pl.ds(f * 16, 16)] = accs[f]

        def pair(m2, carry):
            q0 = 2 * m2
            q1 = q0 + 1
            scan_query(q0, 0)
            fire_dma(0)
            wait_dma(0)
            acc_out(q0, 0)
            scan_query(q1, 1)
            fire_dma(1)
            wait_dma(1)
            acc_out(q1, 1)
            return carry

        lax.fori_loop(0, QW // 2, pair, jnp.array(0, _I32))
        pltpu.sync_copy(outv, out_hbm.at[pl.ds(w * QW, QW)])

    return ballmax


def _mid_body(sm_ref, qx_ref, qy_ref, b1_ref, w1px_ref, w1py_ref,
              w2a_ref, w2px_ref, w2py_ref, a2_ref):
    sm = sm_ref[0]
    qx = qx_ref[0]
    qy = qy_ref[0]
    x1 = sm + b1_ref[...] - qx * w1px_ref[...] - qy * w1py_ref[...]
    a2 = (jnp.dot(x1, w2a_ref[...], preferred_element_type=_F32)
          + qx * w2px_ref[...] + qy * w2py_ref[...])
    a2_ref[0] = a2


def _mid(sm, qx, qy, b1, w1px, w1py, w2a, w2px, w2py):
    B, Nq, _ = sm.shape
    bs3 = lambda d: pl.BlockSpec((1, Nq, d), lambda i: (i, 0, 0))
    ws = lambda s: pl.BlockSpec(s, lambda i: (0,) * len(s))
    return pl.pallas_call(
        _mid_body,
        grid=(B,),
        in_specs=[bs3(64), bs3(1), bs3(1), ws((1, 64)), ws((1, 64)),
                  ws((1, 64)), ws((64, 128)), ws((1, 128)), ws((1, 128))],
        out_specs=bs3(128),
        out_shape=jax.ShapeDtypeStruct((B, Nq, 128), _F32),
    )(sm, qx, qy, b1, w1px, w1py, w2a, w2px, w2py)


def _fin_body(sm_ref, qx_ref, qy_ref, b2_ref, w2px_ref, w2py_ref,
              w3a_ref, w3px_ref, w3py_ref, b3_ref, g_ref):
    sm = sm_ref[0]
    qx = qx_ref[0]
    qy = qy_ref[0]
    x2 = sm + b2_ref[...] - qx * w2px_ref[...] - qy * w2py_ref[...]
    g = (jnp.dot(x2, w3a_ref[...], preferred_element_type=_F32)
         + qx * w3px_ref[...] + qy * w3py_ref[...] + b3_ref[...])
    g_ref[0] = jnp.max(g, axis=0, keepdims=True)


def _fin(sm, qx, qy, b2, w2px, w2py, w3a, w3px, w3py, b3):
    B, Nq, _ = sm.shape
    bs3 = lambda d: pl.BlockSpec((1, Nq, d), lambda i: (i, 0, 0))
    ws = lambda s: pl.BlockSpec(s, lambda i: (0,) * len(s))
    return pl.pallas_call(
        _fin_body,
        grid=(B,),
        in_specs=[bs3(128), bs3(1), bs3(1), ws((1, 128)), ws((1, 128)),
                  ws((1, 128)), ws((128, 1024)), ws((1, 1024)),
                  ws((1, 1024)), ws((1, 1024))],
        out_specs=pl.BlockSpec((1, 1, 1024), lambda i: (i, 0, 0)),
        out_shape=jax.ShapeDtypeStruct((B, 1, 1024), _F32),
    )(sm, qx, qy, b2, w2px, w2py, w3a, w3px, w3py, b3)


def kernel(x, zones_ids, lf_W1, lf_b1, lf_W2, lf_b2, W1, b1, W2, b2, W3, b3):
    B, N, _ = x.shape
    n1 = N // 2
    n2 = n1 // 4
    px = x[:, :, 0]
    py = x[:, :, 1]
    r1 = lambda v: v.reshape(1, -1)

    local, a1 = _prep(x, zones_ids, lf_W1, r1(lf_b1), lf_W2, r1(lf_b2),
                      W1[:64], r1(W1[64]), r1(W1[65]), r1(W1[66]))
    qx1, qy1 = _fps(px, py, n1)
    sm1 = _make_ballmax(B, N, n1, 64, 0.25, 128)(
        px, py, qx1, qy1, a1.reshape(B * N, 128))
    a2 = _mid(sm1.reshape(B, n1, 64), qx1[..., None], qy1[..., None],
              r1(b1), r1(W1[65]), r1(W1[66]), W2[:64], r1(W2[64]), r1(W2[65]))
    qx2, qy2 = _fps(qx1, qy1, n2)
    sm2 = _make_ballmax(B, n1, n2, 128, 1.0, 128)(
        qx1, qy1, qx2, qy2, a2.reshape(B * n1, 128))
    gfeat = _fin(sm2.reshape(B, n2, 128), qx2[..., None], qy2[..., None],
                 r1(b2), r1(W2[64]), r1(W2[65]), W3[:128], r1(W3[128]),
                 r1(W3[129]), r1(b3))
    return local, gfeat.reshape(B, 1024)


# speedup vs baseline: 53.9408x; 1.2976x over previous
import functools

import jax
import jax.numpy as jnp
from jax import lax
from jax.experimental import pallas as pl
from jax.experimental.pallas import tpu as pltpu
from jax.experimental.pallas import tpu_sc as plsc

_F32 = jnp.float32
_I32 = jnp.int32


def _prep_body(x_ref, z_ref, wa_ref, ba_ref, wb_ref, bb_ref, w1a_ref,
               w1z_ref, w1px_ref, w1py_ref, local_ref, a1_ref):
    xr = x_ref[0]
    px = xr[:, 0:1]
    py = xr[:, 1:2]
    zc = z_ref[0]
    h = jnp.tanh(px * wa_ref[0:1] + py * wa_ref[1:2] + ba_ref[...])
    local = jnp.tanh(
        jnp.dot(h, wb_ref[...], preferred_element_type=_F32) + bb_ref[...])
    a1 = (jnp.dot(local, w1a_ref[...], preferred_element_type=_F32)
          + zc * w1z_ref[...] + px * w1px_ref[...] + py * w1py_ref[...])
    local_ref[0] = local
    a1_ref[0] = a1


def _prep(x, z, lfW1, lfb1, lfW2, lfb2, w1a, w1z, w1px, w1py):
    B, N, _ = x.shape
    bs3 = lambda d: pl.BlockSpec((1, N, d), lambda i: (i, 0, 0))
    ws = lambda s: pl.BlockSpec(s, lambda i: (0,) * len(s))
    return pl.pallas_call(
        _prep_body,
        grid=(B,),
        in_specs=[bs3(2), bs3(1), ws((2, 64)), ws((1, 64)), ws((64, 64)),
                  ws((1, 64)), ws((64, 64)), ws((1, 64)), ws((1, 64)),
                  ws((1, 64))],
        out_specs=[bs3(64), bs3(64)],
        out_shape=[jax.ShapeDtypeStruct((B, N, 64), _F32),
                   jax.ShapeDtypeStruct((B, N, 64), _F32)],
    )(x, z, lfW1, lfb1, lfW2, lfb2, w1a, w1z, w1px, w1py)


def _fps_phase(px, py, n_s, qx_ref, qy_ref):
    B, N = px.shape
    iota = lax.broadcasted_iota(_I32, (B, N), 1)
    iota_b = lax.broadcasted_iota(_I32, (B, 128), 1)

    def step(t, carry):
        dists, fidx, qxb, qyb = carry
        onehot = iota == fidx
        fx = jnp.sum(jnp.where(onehot, px, 0.0), axis=1, keepdims=True)
        fy = jnp.sum(jnp.where(onehot, py, 0.0), axis=1, keepdims=True)
        slot = iota_b == t
        qxb = jnp.where(slot, fx, qxb)
        qyb = jnp.where(slot, fy, qyb)
        dx = px - fx
        dy = py - fy
        dists = jnp.minimum(dists, dx * dx + dy * dy)
        m = jnp.max(dists, axis=1, keepdims=True)
        cand = jnp.where(dists == m, iota, N)
        fidx = jnp.min(cand, axis=1, keepdims=True)
        return dists, fidx, qxb, qyb

    dists = jnp.full((B, N), 1e10, _F32)
    fidx = jnp.zeros((B, 1), _I32)
    zb = jnp.zeros((B, 128), _F32)
    for blk in range(n_s // 128):
        dists, fidx, qxb, qyb = lax.fori_loop(
            0, 128, step, (dists, fidx, zb, zb))
        qx_ref[:, blk * 128:(blk + 1) * 128] = qxb
        qy_ref[:, blk * 128:(blk + 1) * 128] = qyb


def _fps_body(n_s, px_ref, py_ref, qx_ref, qy_ref):
    _fps_phase(px_ref[...], py_ref[...], n_s, qx_ref, qy_ref)


def _fps(px, py, n_s):
    B, N = px.shape
    return pl.pallas_call(
        functools.partial(_fps_body, n_s),
        in_specs=[pl.BlockSpec((B, N), lambda: (0, 0))] * 2,
        out_specs=[pl.BlockSpec((B, n_s), lambda: (0, 0))] * 2,
        out_shape=[jax.ShapeDtypeStruct((B, n_s), _F32)] * 2,
    )(px, py)


def _make_ballmax_vmem(B, Nsrc, Nq, F, r2):
    NC, NS = 2, 16
    NW = NC * NS
    QS, FS = 2, 2
    assert NW == B * QS * FS
    QW = Nq // QS
    FH = F // FS
    NFV = FH // 16
    CH = Nsrc // 16
    K = 128
    PK = 128 // FH
    SH = PK.bit_length() - 1
    FSH = FH.bit_length() - 1
    mesh = plsc.VectorSubcoreMesh(core_axis_name="c", subcore_axis_name="s",
                                  num_cores=NC, num_subcores=NS)

    @functools.partial(
        pl.kernel,
        out_type=jax.ShapeDtypeStruct((FS, B * Nq // PK, 128), _F32),
        mesh=mesh,
        compiler_params=pltpu.CompilerParams(needs_layout_passes=False),
        scratch_types=[
            pltpu.VMEM((Nsrc,), _F32), pltpu.VMEM((Nsrc,), _F32),
            pltpu.VMEM((QW,), _F32), pltpu.VMEM((QW,), _F32),
            pltpu.VMEM((K + 16,), _I32), pltpu.VMEM((Nsrc // PK, 128), _F32),
            pltpu.VMEM((QW // PK, 128), _F32),
        ],
    )
    def ballmax(sx_hbm, sy_hbm, qx_hbm, qy_hbm, astk_hbm, out_hbm,
                sxv, syv, qxv, qyv, idxl, tabv, outv):
        c = lax.axis_index("c")
        s = lax.axis_index("s")
        w = s * NC + c
        b = w // (QS * FS)
        r = w % (QS * FS)
        qs = r // FS
        h = r % FS
        qoff = qs * QW
        pltpu.sync_copy(sx_hbm.at[b], sxv)
        pltpu.sync_copy(sy_hbm.at[b], syv)
        pltpu.sync_copy(qx_hbm.at[b, pl.ds(qoff, QW)], qxv)
        pltpu.sync_copy(qy_hbm.at[b, pl.ds(qoff, QW)], qyv)
        pltpu.sync_copy(astk_hbm.at[h, pl.ds(b * (Nsrc // PK), Nsrc // PK)],
                        tabv)
        lane = lax.broadcasted_iota(_I32, (16,), 0)

        def per_query(q, carry):
            qsplat = jnp.full((16,), q, _I32)
            qxs = plsc.load_gather(qxv, [qsplat])
            qys = plsc.load_gather(qyv, [qsplat])

            def cond(st):
                ch, cnt = st
                return jnp.logical_and(ch < CH, cnt < K)

            def body(st):
                ch, cnt = st
                off = ch * 16
                sxc = sxv[pl.ds(off, 16)]
                syc = syv[pl.ds(off, 16)]
                dx = sxc - qxs
                dy = syc - qys
                d2 = dx * dx + dy * dy
                m = d2 <= r2
                plsc.store_compressed(idxl.at[pl.ds(cnt, 16)], lane + off,
                                      mask=m)
                inc = plsc.all_reduce_population_count(m)[0]
                return ch + jnp.array(1, _I32), cnt + inc

            _, cnt = lax.while_loop(
                cond, body, (jnp.array(0, _I32), jnp.array(0, _I32)))

            @pl.when(cnt < K)
            def _pad():
                v0 = plsc.load_gather(idxl, [jnp.zeros((16,), _I32)])
                for c8 in range(K // 16):
                    posn = lane + (c8 * 16)
                    plsc.store_scatter(idxl, [posn], v0, mask=posn >= cnt)

            gdn = lax.GatherDimensionNumbers(
                offset_dims=(), collapsed_slice_dims=(0,),
                start_index_map=(0,))

            def acc_step(k16, accs):
                jvec = idxl[pl.ds(k16 * 16, 16)]
                for u in range(16):
                    jsplat = lax.gather(
                        jvec, jnp.full((16, 1), u, _I32), gdn, (1,),
                        mode=lax.GatherScatterMode.PROMISE_IN_BOUNDS)
                    jrow = lax.shift_right_logical(jsplat, SH)
                    jcol = lax.shift_left(jnp.bitwise_and(jsplat, PK - 1), FSH)
                    accs = tuple(
                        jnp.maximum(
                            accs[f],
                            plsc.load_gather(tabv,
                                             [jrow, jcol + (lane + 16 * f)]))
                        for f in range(NFV))
                return accs

            accs = lax.fori_loop(
                0, K // 16, acc_step,
                tuple(jnp.full((16,), -3.0e38, _F32) for _ in range(NFV)))
            qrow = jnp.full((16,), lax.shift_right_logical(q, SH), _I32)
            qcol = lax.shift_left(jnp.bitwise_and(q, PK - 1), FSH)
            for f in range(NFV):
                plsc.store_scatter(outv, [qrow, qcol + (lane + 16 * f)],
                                   accs[f])
            return carry

        lax.fori_loop(0, QW, per_query, jnp.array(0, _I32))
        orow = pl.multiple_of((b * Nq + qoff) // PK, QW // PK)
        pltpu.sync_copy(outv, out_hbm.at[h, pl.ds(orow, QW // PK)])

    return ballmax


def _mid_body(sm_ref, qx_ref, qy_ref, b1_ref, w1px_ref, w1py_ref,
              w2a_ref, w2px_ref, w2py_ref, a2_ref):
    sm = sm_ref[0]
    qx = qx_ref[0]
    qy = qy_ref[0]
    x1 = sm + b1_ref[...] - qx * w1px_ref[...] - qy * w1py_ref[...]
    a2 = (jnp.dot(x1, w2a_ref[...], preferred_element_type=_F32)
          + qx * w2px_ref[...] + qy * w2py_ref[...])
    a2_ref[0] = a2


def _mid(sm, qx, qy, b1, w1px, w1py, w2a, w2px, w2py):
    B, Nq, _ = sm.shape
    bs3 = lambda d: pl.BlockSpec((1, Nq, d), lambda i: (i, 0, 0))
    ws = lambda s: pl.BlockSpec(s, lambda i: (0,) * len(s))
    return pl.pallas_call(
        _mid_body,
        grid=(B,),
        in_specs=[bs3(64), bs3(1), bs3(1), ws((1, 64)), ws((1, 64)),
                  ws((1, 64)), ws((64, 128)), ws((1, 128)), ws((1, 128))],
        out_specs=bs3(128),
        out_shape=jax.ShapeDtypeStruct((B, Nq, 128), _F32),
    )(sm, qx, qy, b1, w1px, w1py, w2a, w2px, w2py)


def _fin_body(sm_ref, qx_ref, qy_ref, b2_ref, w2px_ref, w2py_ref,
              w3a_ref, w3px_ref, w3py_ref, b3_ref, g_ref):
    sm = sm_ref[0]
    qx = qx_ref[0]
    qy = qy_ref[0]
    x2 = sm + b2_ref[...] - qx * w2px_ref[...] - qy * w2py_ref[...]
    g = (jnp.dot(x2, w3a_ref[...], preferred_element_type=_F32)
         + qx * w3px_ref[...] + qy * w3py_ref[...] + b3_ref[...])
    g_ref[0] = jnp.max(g, axis=0, keepdims=True)


def _fin(sm, qx, qy, b2, w2px, w2py, w3a, w3px, w3py, b3):
    B, Nq, _ = sm.shape
    bs3 = lambda d: pl.BlockSpec((1, Nq, d), lambda i: (i, 0, 0))
    ws = lambda s: pl.BlockSpec(s, lambda i: (0,) * len(s))
    return pl.pallas_call(
        _fin_body,
        grid=(B,),
        in_specs=[bs3(128), bs3(1), bs3(1), ws((1, 128)), ws((1, 128)),
                  ws((1, 128)), ws((128, 1024)), ws((1, 1024)),
                  ws((1, 1024)), ws((1, 1024))],
        out_specs=pl.BlockSpec((1, 1, 1024), lambda i: (i, 0, 0)),
        out_shape=jax.ShapeDtypeStruct((B, 1, 1024), _F32),
    )(sm, qx, qy, b2, w2px, w2py, w3a, w3px, w3py, b3)


def kernel(x, zones_ids, lf_W1, lf_b1, lf_W2, lf_b2, W1, b1, W2, b2, W3, b3):
    B, N, _ = x.shape
    n1 = N // 2
    n2 = n1 // 4
    px = x[:, :, 0]
    py = x[:, :, 1]
    r1 = lambda v: v.reshape(1, -1)

    local, a1 = _prep(x, zones_ids, lf_W1, r1(lf_b1), lf_W2, r1(lf_b2),
                      W1[:64], r1(W1[64]), r1(W1[65]), r1(W1[66]))
    qx1, qy1 = _fps(px, py, n1)
    a1r = a1.reshape(B * N, 64)
    astk = jnp.stack([a1r[:, :32].reshape(B * N // 4, 128),
                      a1r[:, 32:].reshape(B * N // 4, 128)], axis=0)
    sm1h = _make_ballmax_vmem(B, N, n1, 64, 0.25)(
        px, py, qx1, qy1, astk)
    sm1 = jnp.concatenate([sm1h[0].reshape(B * n1, 32),
                           sm1h[1].reshape(B * n1, 32)], axis=1)
    a2 = _mid(sm1.reshape(B, n1, 64), qx1[..., None], qy1[..., None],
              r1(b1), r1(W1[65]), r1(W1[66]), W2[:64], r1(W2[64]), r1(W2[65]))
    qx2, qy2 = _fps(qx1, qy1, n2)
    a2r = a2.reshape(B * n1, 128)
    astk2 = jnp.stack([a2r[:, :64].reshape(B * n1 // 2, 128),
                       a2r[:, 64:].reshape(B * n1 // 2, 128)], axis=0)
    sm2h = _make_ballmax_vmem(B, n1, n2, 128, 1.0)(
        qx1, qy1, qx2, qy2, astk2)
    sm2 = jnp.concatenate([sm2h[0].reshape(B * n2, 64),
                           sm2h[1].reshape(B * n2, 64)], axis=1)
    gfeat = _fin(sm2.reshape(B, n2, 128), qx2[..., None], qy2[..., None],
                 r1(b2), r1(W2[64]), r1(W2[65]), W3[:128], r1(W3[128]),
                 r1(W3[129]), r1(b3))
    return local, gfeat.reshape(B, 1024)
